# baseline (speedup 1.0000x reference)
import jax
import jax.numpy as jnp
from jax import lax
from jax.experimental import pallas as pl

B, S, H, I = 16384, 50, 64, 64
BB = 256


def _body(pos_ref, sub_ref, sh_ref, sc_ref, w_ref, b_ref,
          hout_ref, cout_ref, shout_ref, scout_ref):
    pos = pos_ref[...]
    x3h = sh_ref[...]
    x3c = sc_ref[...]
    s_iota = lax.broadcasted_iota(jnp.int32, (S, 1, 1), 0)
    pm = pos[None, :, :]
    maskg = s_iota == pm
    h = jnp.sum(jnp.where(maskg, x3h, 0.0), axis=0)
    c = jnp.sum(jnp.where(maskg, x3c, 0.0), axis=0)
    x = jnp.concatenate([sub_ref[...], h], axis=0)
    gates = jnp.dot(w_ref[...], x, preferred_element_type=jnp.float32)
    gates = gates + b_ref[...]
    i_g = jax.nn.sigmoid(gates[0:H])
    f_g = jax.nn.sigmoid(gates[H:2 * H])
    g_g = jnp.tanh(gates[2 * H:3 * H])
    o_g = jax.nn.sigmoid(gates[3 * H:4 * H])
    c_new = f_g * c + i_g * g_g
    h_new = o_g * jnp.tanh(c_new)
    hout_ref[...] = h_new
    cout_ref[...] = c_new
    masks = s_iota == pm + 1
    shout_ref[...] = jnp.where(masks, h_new[None], x3h)
    scout_ref[...] = jnp.where(masks, c_new[None], x3c)


def kernel(subword, stack_hidden, stack_cell, idx, pos,
           weight_ih, weight_hh, bias_ih, bias_hh):
    del idx
    subt = subword.T
    sht = jnp.transpose(stack_hidden, (1, 2, 0))
    sct = jnp.transpose(stack_cell, (1, 2, 0))
    w = jnp.concatenate([weight_ih, weight_hh], axis=1)
    bias = (bias_ih + bias_hh).reshape(4 * H, 1)
    pos2d = pos.reshape(1, B)
    grid = (B // BB,)
    out = pl.pallas_call(
        _body,
        grid=grid,
        in_specs=[
            pl.BlockSpec((1, BB), lambda i: (0, i)),
            pl.BlockSpec((I, BB), lambda i: (0, i)),
            pl.BlockSpec((S, H, BB), lambda i: (0, 0, i)),
            pl.BlockSpec((S, H, BB), lambda i: (0, 0, i)),
            pl.BlockSpec((4 * H, I + H), lambda i: (0, 0)),
            pl.BlockSpec((4 * H, 1), lambda i: (0, 0)),
        ],
        out_specs=[
            pl.BlockSpec((H, BB), lambda i: (0, i)),
            pl.BlockSpec((H, BB), lambda i: (0, i)),
            pl.BlockSpec((S, H, BB), lambda i: (0, 0, i)),
            pl.BlockSpec((S, H, BB), lambda i: (0, 0, i)),
        ],
        out_shape=[
            jax.ShapeDtypeStruct((H, B), jnp.float32),
            jax.ShapeDtypeStruct((H, B), jnp.float32),
            jax.ShapeDtypeStruct((S, H, B), jnp.float32),
            jax.ShapeDtypeStruct((S, H, B), jnp.float32),
        ],
    )(pos2d, subt, sht, sct, w, bias)
    h_t, c_t, sh_t, sc_t = out
    return (h_t.T, c_t.T,
            jnp.transpose(sh_t, (2, 0, 1)),
            jnp.transpose(sc_t, (2, 0, 1)))


# speedup vs baseline: 1.0282x; 1.0282x over previous
import jax
import jax.numpy as jnp
from jax import lax
from jax.experimental import pallas as pl

B, S, H, I = 16384, 50, 64, 64
BB = 512


def _body(pos_ref, sub_ref, sh_ref, sc_ref, w_ref, b_ref,
          hout_ref, cout_ref, shout_ref, scout_ref):
    pos = pos_ref[...]
    x3h = sh_ref[...]
    x3c = sc_ref[...]
    s_iota = lax.broadcasted_iota(jnp.int32, (S, 1, 1), 0)
    pm = pos[None, :, :]
    maskg = s_iota == pm
    h = jnp.sum(jnp.where(maskg, x3h, 0.0), axis=0)
    c = jnp.sum(jnp.where(maskg, x3c, 0.0), axis=0)
    x = jnp.concatenate([sub_ref[...], h], axis=0)
    gates = jnp.dot(w_ref[...], x, preferred_element_type=jnp.float32)
    gates = gates + b_ref[...]
    i_g = jax.nn.sigmoid(gates[0:H])
    f_g = jax.nn.sigmoid(gates[H:2 * H])
    g_g = jnp.tanh(gates[2 * H:3 * H])
    o_g = jax.nn.sigmoid(gates[3 * H:4 * H])
    c_new = f_g * c + i_g * g_g
    h_new = o_g * jnp.tanh(c_new)
    hout_ref[...] = h_new
    cout_ref[...] = c_new
    masks = s_iota == pm + 1
    shout_ref[...] = jnp.where(masks, h_new[None], x3h)
    scout_ref[...] = jnp.where(masks, c_new[None], x3c)


def kernel(subword, stack_hidden, stack_cell, idx, pos,
           weight_ih, weight_hh, bias_ih, bias_hh):
    del idx
    subt = subword.T
    sht = jnp.transpose(stack_hidden, (1, 2, 0))
    sct = jnp.transpose(stack_cell, (1, 2, 0))
    w = jnp.concatenate([weight_ih, weight_hh], axis=1)
    bias = (bias_ih + bias_hh).reshape(4 * H, 1)
    pos2d = pos.reshape(1, B)
    grid = (B // BB,)
    out = pl.pallas_call(
        _body,
        grid=grid,
        in_specs=[
            pl.BlockSpec((1, BB), lambda i: (0, i)),
            pl.BlockSpec((I, BB), lambda i: (0, i)),
            pl.BlockSpec((S, H, BB), lambda i: (0, 0, i)),
            pl.BlockSpec((S, H, BB), lambda i: (0, 0, i)),
            pl.BlockSpec((4 * H, I + H), lambda i: (0, 0)),
            pl.BlockSpec((4 * H, 1), lambda i: (0, 0)),
        ],
        out_specs=[
            pl.BlockSpec((H, BB), lambda i: (0, i)),
            pl.BlockSpec((H, BB), lambda i: (0, i)),
            pl.BlockSpec((S, H, BB), lambda i: (0, 0, i)),
            pl.BlockSpec((S, H, BB), lambda i: (0, 0, i)),
        ],
        out_shape=[
            jax.ShapeDtypeStruct((H, B), jnp.float32),
            jax.ShapeDtypeStruct((H, B), jnp.float32),
            jax.ShapeDtypeStruct((S, H, B), jnp.float32),
            jax.ShapeDtypeStruct((S, H, B), jnp.float32),
        ],
    )(pos2d, subt, sht, sct, w, bias)
    h_t, c_t, sh_t, sc_t = out
    return (h_t.T, c_t.T,
            jnp.transpose(sh_t, (2, 0, 1)),
            jnp.transpose(sc_t, (2, 0, 1)))
